# baseline (speedup 1.0000x reference)
import functools

import jax
import jax.numpy as jnp
from jax import lax
from jax.experimental import pallas as pl
from jax.experimental.pallas import tpu as pltpu
from jax.experimental.pallas import tpu_sc as plsc

ENT = 1000000
REL = 1000
D = 64
MARGIN = 1.0
B = 4096
NEG = 25

NC = 2
NS = 16
L = 16
NW = NC * NS

PAIRS = NEG + 1
B_PER_W = B // NW
E_PER_C = 4
CHUNK = E_PER_C * PAIRS
NCHUNK = B_PER_W // E_PER_C
NCPAD = NCHUNK + 1
KD = D // L


def _body(h_hbm, t_hbm, r_hbm, ent_hbm, rel_hbm, out_hbm,
          h_idx, t_idx, r_idx,
          bh_a, bt_a, br_a, bh_b, bt_b, br_b, loss_v,
          semh_a, semt_a, semr_a, semh_b, semt_b, semr_b):
    wid = lax.axis_index("s") * NC + lax.axis_index("c")
    iota = lax.iota(jnp.int32, L)

    pltpu.sync_copy(h_hbm.at[wid], h_idx)
    pltpu.sync_copy(t_hbm.at[wid], t_idx)
    pltpu.sync_copy(r_hbm.at[wid], r_idx)

    def row_copies(c, e, idx_ref, table, buf, sem):
        eoff = e * PAIRS
        v0 = idx_ref.at[c][pl.ds(eoff, L)]
        v1 = idx_ref.at[c][pl.ds(eoff + PAIRS - L, L)]
        for j in range(PAIRS):
            row = v0[j] if j < L else v1[j - (PAIRS - L)]
            pltpu.async_copy(table.at[pl.ds(row, 1)],
                             buf.at[pl.ds(eoff + j, 1)], sem)

    def issue(c, bh, bt, br, semh, semt, semr):
        def issue_e(e, carry):
            row_copies(c, e, h_idx, ent_hbm, bh, semh)
            row_copies(c, e, t_idx, ent_hbm, bt, semt)
            row_copies(c, e, r_idx, rel_hbm, br, semr)
            return carry

        lax.fori_loop(0, E_PER_C, issue_e, jnp.int32(0))

    def drain(bh, bt, br, semh, semt, semr):
        for sem, buf in ((semh, bh), (semt, bt), (semr, br)):
            for _ in range(CHUNK):
                pltpu.make_async_copy(ent_hbm.at[pl.ds(0, 1)],
                                      buf.at[pl.ds(0, 1)], sem).wait()

    def compute(loss, bh, bt, br):
        def pair_acc(acc, p):
            for k in range(KD):
                sl = pl.ds(k * L, L)
                acc = acc + jnp.abs(bh[p, sl] - bt[p, sl] + br[p, sl])
            return acc

        def e_body(e, loss):
            base = e * PAIRS
            snv = jnp.zeros((L,), jnp.float32)
            for j in range(NEG):
                snv = pair_acc(snv, base + j)
            spv = pair_acc(jnp.zeros((L,), jnp.float32), base + NEG)
            sn = lax.reduce_sum_p.bind(snv, axes=(0,))
            sp = lax.reduce_sum_p.bind(spv, axes=(0,))
            return loss + jnp.maximum(sp - sn * (1.0 / NEG) + MARGIN, 0.0)

        return lax.fori_loop(0, E_PER_C, e_body, loss)

    a_bufs = (bh_a, bt_a, br_a, semh_a, semt_a, semr_a)
    b_bufs = (bh_b, bt_b, br_b, semh_b, semt_b, semr_b)

    issue(0, *a_bufs)

    def body2(k, loss):
        c = 2 * k
        issue(c + 1, *b_bufs)
        drain(*a_bufs)
        loss = compute(loss, bh_a, bt_a, br_a)
        issue(c + 2, *a_bufs)
        drain(*b_bufs)
        return compute(loss, bh_b, bt_b, br_b)

    loss = lax.fori_loop(0, NCHUNK // 2, body2, jnp.float32(0.0))
    drain(*a_bufs)

    loss_v[...] = jnp.where(iota == 0, loss, 0.0)
    pltpu.sync_copy(loss_v, out_hbm.at[wid])


def kernel(pos_h, pos_t, pos_r, neg_h, neg_t, neg_r, ent_embeddings, rel_embeddings):
    def prep(neg, pos):
        x = jnp.concatenate([neg, pos], axis=1).reshape(NW, NCHUNK, CHUNK)
        return jnp.pad(x, ((0, 0), (0, 1), (0, 0)))

    h3 = prep(neg_h, pos_h)
    t3 = prep(neg_t, pos_t)
    r3 = prep(neg_r, pos_r)

    run = functools.partial(
        pl.kernel,
        mesh=plsc.VectorSubcoreMesh(core_axis_name="c", subcore_axis_name="s"),
        compiler_params=pltpu.CompilerParams(
            needs_layout_passes=False, use_tc_tiling_on_sc=True),
        out_type=jax.ShapeDtypeStruct((NW, L), jnp.float32),
        scratch_types=[
            pltpu.VMEM((NCPAD, CHUNK), jnp.int32),
            pltpu.VMEM((NCPAD, CHUNK), jnp.int32),
            pltpu.VMEM((NCPAD, CHUNK), jnp.int32),
            pltpu.VMEM((CHUNK, D), jnp.float32),
            pltpu.VMEM((CHUNK, D), jnp.float32),
            pltpu.VMEM((CHUNK, D), jnp.float32),
            pltpu.VMEM((CHUNK, D), jnp.float32),
            pltpu.VMEM((CHUNK, D), jnp.float32),
            pltpu.VMEM((CHUNK, D), jnp.float32),
            pltpu.VMEM((L,), jnp.float32),
            pltpu.SemaphoreType.DMA,
            pltpu.SemaphoreType.DMA,
            pltpu.SemaphoreType.DMA,
            pltpu.SemaphoreType.DMA,
            pltpu.SemaphoreType.DMA,
            pltpu.SemaphoreType.DMA,
        ],
    )(_body)

    partials = run(h3, t3, r3, ent_embeddings, rel_embeddings)
    return jnp.sum(partials)


# speedup vs baseline: 1.2128x; 1.2128x over previous
import functools

import jax
import jax.numpy as jnp
from jax import lax
from jax.experimental import pallas as pl
from jax.experimental.pallas import tpu as pltpu
from jax.experimental.pallas import tpu_sc as plsc

ENT = 1000000
REL = 1000
D = 64
MARGIN = 1.0
B = 4096
NEG = 25

NC = 2
NS = 16
L = 16
NW = NC * NS

PAIRS = NEG + 1
B_PER_W = B // NW
E_PER_C = 2
CHUNK = E_PER_C * PAIRS
NCHUNK = B_PER_W // E_PER_C
NCPAD = NCHUNK + 1
KD = D // L


def _body(h_hbm, t_hbm, r_hbm, ent_hbm, rel_hbm, out_hbm,
          h_idx, t_idx, r_idx,
          bh_a, bt_a, br_a, bh_b, bt_b, br_b, loss_v,
          semh_a, semt_a, semr_a, semh_b, semt_b, semr_b):
    wid = lax.axis_index("s") * NC + lax.axis_index("c")
    iota = lax.iota(jnp.int32, L)

    pltpu.sync_copy(h_hbm.at[wid], h_idx)
    pltpu.sync_copy(t_hbm.at[wid], t_idx)
    pltpu.sync_copy(r_hbm.at[wid], r_idx)

    def row_copies(c, e, idx_ref, table, buf, sem):
        eoff = e * PAIRS
        v0 = idx_ref.at[c][pl.ds(eoff, L)]
        v1 = idx_ref.at[c][pl.ds(eoff + PAIRS - L, L)]
        for j in range(PAIRS):
            row = v0[j] if j < L else v1[j - (PAIRS - L)]
            pltpu.async_copy(table.at[pl.ds(row, 1)],
                             buf.at[pl.ds(eoff + j, 1)], sem)

    def issue(c, bh, bt, br, semh, semt, semr):
        def issue_e(e, carry):
            row_copies(c, e, h_idx, ent_hbm, bh, semh)
            row_copies(c, e, t_idx, ent_hbm, bt, semt)
            row_copies(c, e, r_idx, rel_hbm, br, semr)
            return carry

        lax.fori_loop(0, E_PER_C, issue_e, jnp.int32(0))

    def drain(bh, bt, br, semh, semt, semr):
        for sem, buf in ((semh, bh), (semt, bt), (semr, br)):
            for _ in range(CHUNK):
                pltpu.make_async_copy(ent_hbm.at[pl.ds(0, 1)],
                                      buf.at[pl.ds(0, 1)], sem).wait()

    def compute(loss, bh, bt, br):
        def pair_acc(acc, p):
            for k in range(KD):
                sl = pl.ds(k * L, L)
                acc = acc + jnp.abs(bh[p, sl] - bt[p, sl] + br[p, sl])
            return acc

        def e_body(e, loss):
            base = e * PAIRS
            snv = jnp.zeros((L,), jnp.float32)
            for j in range(NEG):
                snv = pair_acc(snv, base + j)
            spv = pair_acc(jnp.zeros((L,), jnp.float32), base + NEG)
            sn = lax.reduce_sum_p.bind(snv, axes=(0,))
            sp = lax.reduce_sum_p.bind(spv, axes=(0,))
            return loss + jnp.maximum(sp - sn * (1.0 / NEG) + MARGIN, 0.0)

        return lax.fori_loop(0, E_PER_C, e_body, loss)

    a_bufs = (bh_a, bt_a, br_a, semh_a, semt_a, semr_a)
    b_bufs = (bh_b, bt_b, br_b, semh_b, semt_b, semr_b)

    issue(0, *a_bufs)

    def body2(k, loss):
        c = 2 * k
        issue(c + 1, *b_bufs)
        drain(*a_bufs)
        loss = compute(loss, bh_a, bt_a, br_a)
        issue(c + 2, *a_bufs)
        drain(*b_bufs)
        return compute(loss, bh_b, bt_b, br_b)

    loss = lax.fori_loop(0, NCHUNK // 2, body2, jnp.float32(0.0))
    drain(*a_bufs)

    loss_v[...] = jnp.where(iota == 0, loss, 0.0)
    pltpu.sync_copy(loss_v, out_hbm.at[wid])


def kernel(pos_h, pos_t, pos_r, neg_h, neg_t, neg_r, ent_embeddings, rel_embeddings):
    def prep(neg, pos):
        x = jnp.concatenate([neg, pos], axis=1).reshape(NW, NCHUNK, CHUNK)
        return jnp.pad(x, ((0, 0), (0, 1), (0, 0)))

    h3 = prep(neg_h, pos_h)
    t3 = prep(neg_t, pos_t)
    r3 = prep(neg_r, pos_r)

    run = functools.partial(
        pl.kernel,
        mesh=plsc.VectorSubcoreMesh(core_axis_name="c", subcore_axis_name="s"),
        compiler_params=pltpu.CompilerParams(
            needs_layout_passes=False, use_tc_tiling_on_sc=True),
        out_type=jax.ShapeDtypeStruct((NW, L), jnp.float32),
        scratch_types=[
            pltpu.VMEM((NCPAD, CHUNK), jnp.int32),
            pltpu.VMEM((NCPAD, CHUNK), jnp.int32),
            pltpu.VMEM((NCPAD, CHUNK), jnp.int32),
            pltpu.VMEM((CHUNK, D), jnp.float32),
            pltpu.VMEM((CHUNK, D), jnp.float32),
            pltpu.VMEM((CHUNK, D), jnp.float32),
            pltpu.VMEM((CHUNK, D), jnp.float32),
            pltpu.VMEM((CHUNK, D), jnp.float32),
            pltpu.VMEM((CHUNK, D), jnp.float32),
            pltpu.VMEM((L,), jnp.float32),
            pltpu.SemaphoreType.DMA,
            pltpu.SemaphoreType.DMA,
            pltpu.SemaphoreType.DMA,
            pltpu.SemaphoreType.DMA,
            pltpu.SemaphoreType.DMA,
            pltpu.SemaphoreType.DMA,
        ],
    )(_body)

    partials = run(h3, t3, r3, ent_embeddings, rel_embeddings)
    return jnp.sum(partials)


# speedup vs baseline: 1.5643x; 1.2897x over previous
import functools

import jax
import jax.numpy as jnp
from jax import lax
from jax.experimental import pallas as pl
from jax.experimental.pallas import tpu as pltpu
from jax.experimental.pallas import tpu_sc as plsc

ENT = 1000000
REL = 1000
D = 64
MARGIN = 1.0
B = 4096
NEG = 25

NC = 2
NS = 16
L = 16
NW = NC * NS

PAIRS = NEG + 1
B_PER_W = B // NW
E_PER_C = 4
CHUNK = E_PER_C * PAIRS
NCHUNK = B_PER_W // E_PER_C
KD = D // L


def _body(h_hbm, t_hbm, r_hbm, ent_hbm, rel_hbm, out_hbm,
          h_idx, t_idx, r_idx, bh, bt, br, loss_v,
          semh, semt, semr):
    wid = lax.axis_index("s") * NC + lax.axis_index("c")
    iota = lax.iota(jnp.int32, L)

    pltpu.sync_copy(h_hbm.at[wid], h_idx)
    pltpu.sync_copy(t_hbm.at[wid], t_idx)
    pltpu.sync_copy(r_hbm.at[wid], r_idx)

    def row_copies(c, e, idx_ref, table, buf, sem):
        eoff = e * PAIRS
        v0 = idx_ref.at[c][pl.ds(eoff, L)]
        v1 = idx_ref.at[c][pl.ds(eoff + PAIRS - L, L)]
        for j in range(PAIRS):
            row = v0[j] if j < L else v1[j - (PAIRS - L)]
            pltpu.async_copy(table.at[pl.ds(row, 1)],
                             buf.at[pl.ds(eoff + j, 1)], sem)

    def pair_acc(acc, p):
        for k in range(KD):
            sl = pl.ds(k * L, L)
            acc = acc + jnp.abs(bh[p, sl] - bt[p, sl] + br[p, sl])
        return acc

    def chunk_body(c, loss):
        def issue(e, carry):
            row_copies(c, e, h_idx, ent_hbm, bh, semh)
            row_copies(c, e, t_idx, ent_hbm, bt, semt)
            row_copies(c, e, r_idx, rel_hbm, br, semr)
            return carry

        lax.fori_loop(0, E_PER_C, issue, jnp.int32(0))

        def e_body(e, loss):
            for sem in (semh, semt, semr):
                for _ in range(PAIRS):
                    pltpu.make_async_copy(ent_hbm.at[pl.ds(0, 1)],
                                          bh.at[pl.ds(0, 1)], sem).wait()
            base = e * PAIRS
            snv = jnp.zeros((L,), jnp.float32)
            for j in range(NEG):
                snv = pair_acc(snv, base + j)
            spv = pair_acc(jnp.zeros((L,), jnp.float32), base + NEG)
            sn = lax.reduce_sum_p.bind(snv, axes=(0,))
            sp = lax.reduce_sum_p.bind(spv, axes=(0,))
            return loss + jnp.maximum(sp - sn * (1.0 / NEG) + MARGIN, 0.0)

        return lax.fori_loop(0, E_PER_C, e_body, loss)

    loss = lax.fori_loop(0, NCHUNK, chunk_body, jnp.float32(0.0))

    loss_v[...] = jnp.where(iota == 0, loss, 0.0)
    pltpu.sync_copy(loss_v, out_hbm.at[wid])


def kernel(pos_h, pos_t, pos_r, neg_h, neg_t, neg_r, ent_embeddings, rel_embeddings):
    h3 = jnp.concatenate([neg_h, pos_h], axis=1).reshape(NW, NCHUNK, CHUNK)
    t3 = jnp.concatenate([neg_t, pos_t], axis=1).reshape(NW, NCHUNK, CHUNK)
    r3 = jnp.concatenate([neg_r, pos_r], axis=1).reshape(NW, NCHUNK, CHUNK)

    run = functools.partial(
        pl.kernel,
        mesh=plsc.VectorSubcoreMesh(core_axis_name="c", subcore_axis_name="s"),
        compiler_params=pltpu.CompilerParams(
            needs_layout_passes=False, use_tc_tiling_on_sc=True),
        out_type=jax.ShapeDtypeStruct((NW, L), jnp.float32),
        scratch_types=[
            pltpu.VMEM((NCHUNK, CHUNK), jnp.int32),
            pltpu.VMEM((NCHUNK, CHUNK), jnp.int32),
            pltpu.VMEM((NCHUNK, CHUNK), jnp.int32),
            pltpu.VMEM((CHUNK, D), jnp.float32),
            pltpu.VMEM((CHUNK, D), jnp.float32),
            pltpu.VMEM((CHUNK, D), jnp.float32),
            pltpu.VMEM((L,), jnp.float32),
            pltpu.SemaphoreType.DMA,
            pltpu.SemaphoreType.DMA,
            pltpu.SemaphoreType.DMA,
        ],
    )(_body)

    partials = run(h3, t3, r3, ent_embeddings, rel_embeddings)
    return jnp.sum(partials)
